# baseline (speedup 1.0000x reference)
import functools

import jax
import jax.numpy as jnp
from jax import lax
from jax.experimental import pallas as pl
from jax.experimental.pallas import tpu as pltpu
from jax.experimental.pallas import tpu_sc as plsc

N, E, D, DE, H, T, L = 10000, 320000, 128, 16, 128, 32, 5
NC, NS = 2, 16
NW = NC * NS
EPW = E // NW
NP = 10240
RPT = NP // NS

_f32 = jnp.float32


CH = 50
NCHUNK = EPW // CH
SLOTC = 8
NSLOT = NCHUNK // SLOTC
RING = 4


def _spmm_body(f_hbm, src_hbm, dst_hbm, z_hbm, out_hbm, src_sl, dst_sl, rows,
               acc, isem, gsem0, gsem1, gsem2, gsem3, ssem):
    gsem = (gsem0, gsem1, gsem2, gsem3)
    c = lax.axis_index("c")
    s = lax.axis_index("s")
    wid = c * NS + s
    pltpu.sync_copy(src_hbm.at[wid, pl.ds(0, SLOTC)], src_sl.at[0])
    pltpu.sync_copy(dst_hbm.at[wid, pl.ds(0, SLOTC)], dst_sl.at[0])
    pltpu.sync_copy(z_hbm, acc.at[pl.ds(s * RPT, RPT)])
    plsc.subcore_barrier()

    for jj in range(RING - 1):
        pltpu.async_copy(f_hbm.at[src_sl.at[0, jj]], rows.at[jj], gsem[jj])

    def step(j0, carry):
        for u in range(RING):
            j = j0 * RING + u
            g = lax.div(j, SLOTC)
            k = lax.rem(j, SLOTC)
            b = lax.rem(g, 2)

            @pl.when(j >= 1)
            def _(u=u):
                pltpu.make_async_copy(rows.at[(u + RING - 1) % RING],
                                      acc.at[dst_sl.at[0, 0]], ssem).wait()

            @pl.when(jnp.logical_and(k == 0, j + SLOTC < NCHUNK))
            def _(g=g):
                bn = lax.rem(g + 1, 2)
                pltpu.async_copy(src_hbm.at[wid, pl.ds((g + 1) * SLOTC, SLOTC)],
                                 src_sl.at[bn], isem)
                pltpu.async_copy(dst_hbm.at[wid, pl.ds((g + 1) * SLOTC, SLOTC)],
                                 dst_sl.at[bn], isem)

            @pl.when(j + RING - 1 < NCHUNK)
            def _(j=j, u=u):
                j3 = j + RING - 1
                g3 = lax.div(j3, SLOTC)
                k3 = lax.rem(j3, SLOTC)
                b3 = lax.rem(g3, 2)

                @pl.when(k3 == 0)
                def _():
                    pltpu.make_async_copy(
                        src_hbm.at[wid, pl.ds(g3 * SLOTC, SLOTC)],
                        src_sl.at[b3], isem).wait()
                    pltpu.make_async_copy(
                        dst_hbm.at[wid, pl.ds(g3 * SLOTC, SLOTC)],
                        dst_sl.at[b3], isem).wait()

                pltpu.async_copy(f_hbm.at[src_sl.at[b3, k3]],
                                 rows.at[(u + RING - 1) % RING],
                                 gsem[(u + RING - 1) % RING])

            pltpu.make_async_copy(f_hbm.at[src_sl.at[b, k]],
                                  rows.at[u], gsem[u]).wait()
            pltpu.async_copy(rows.at[u], acc.at[dst_sl.at[b, k]], ssem, add=True)
        return carry

    lax.fori_loop(0, NCHUNK // RING, step, 0)
    pltpu.make_async_copy(rows.at[(NCHUNK - 1) % RING],
                          acc.at[dst_sl.at[0, 0]], ssem).wait()
    plsc.subcore_barrier()
    pltpu.sync_copy(acc.at[pl.ds(s * RPT, RPT)], out_hbm.at[c, pl.ds(s * RPT, RPT)])


_spmm_call = functools.partial(
    pl.kernel,
    mesh=plsc.VectorSubcoreMesh(core_axis_name="c", subcore_axis_name="s",
                                num_cores=NC, num_subcores=NS),
    out_type=jax.ShapeDtypeStruct((NC, NP, H), _f32),
    scratch_types=[
        pltpu.VMEM((2, SLOTC, CH), jnp.int32),
        pltpu.VMEM((2, SLOTC, CH), jnp.int32),
        pltpu.VMEM((RING, CH, H), _f32),
        pltpu.VMEM_SHARED((NP, H), _f32),
        pltpu.SemaphoreType.DMA,
        pltpu.SemaphoreType.DMA,
        pltpu.SemaphoreType.DMA,
        pltpu.SemaphoreType.DMA,
        pltpu.SemaphoreType.DMA,
        pltpu.SemaphoreType.DMA,
    ],
)(_spmm_body)


def _ea_body(ea3_hbm, dst_hbm, z_hbm, out_hbm, dst_sl, rows, acc,
             isem, gsem0, gsem1, gsem2, gsem3, ssem):
    gsem = (gsem0, gsem1, gsem2, gsem3)
    c = lax.axis_index("c")
    s = lax.axis_index("s")
    wid = c * NS + s
    pltpu.sync_copy(dst_hbm.at[wid, pl.ds(0, SLOTC)], dst_sl.at[0])
    pltpu.sync_copy(z_hbm, acc.at[pl.ds(s * RPT, RPT)])
    plsc.subcore_barrier()

    for jj in range(RING - 1):
        pltpu.async_copy(ea3_hbm.at[wid * NCHUNK + jj], rows.at[jj], gsem[jj])

    def step(j0, carry):
        for u in range(RING):
            j = j0 * RING + u
            g = lax.div(j, SLOTC)
            k = lax.rem(j, SLOTC)
            b = lax.rem(g, 2)

            @pl.when(j >= 1)
            def _(u=u):
                pltpu.make_async_copy(rows.at[(u + RING - 1) % RING],
                                      acc.at[dst_sl.at[0, 0]], ssem).wait()

            @pl.when(jnp.logical_and(k == 0, j + SLOTC < NCHUNK))
            def _(g=g):
                bn = lax.rem(g + 1, 2)
                pltpu.async_copy(dst_hbm.at[wid, pl.ds((g + 1) * SLOTC, SLOTC)],
                                 dst_sl.at[bn], isem)

            @pl.when(j + RING - 1 < NCHUNK)
            def _(j=j, u=u):
                j3 = j + RING - 1
                g3 = lax.div(j3, SLOTC)
                k3 = lax.rem(j3, SLOTC)
                b3 = lax.rem(g3, 2)

                @pl.when(k3 == 0)
                def _():
                    pltpu.make_async_copy(
                        dst_hbm.at[wid, pl.ds(g3 * SLOTC, SLOTC)],
                        dst_sl.at[b3], isem).wait()

                pltpu.async_copy(ea3_hbm.at[wid * NCHUNK + j3],
                                 rows.at[(u + RING - 1) % RING],
                                 gsem[(u + RING - 1) % RING])

            pltpu.make_async_copy(ea3_hbm.at[wid * NCHUNK + j],
                                  rows.at[u], gsem[u]).wait()
            pltpu.async_copy(rows.at[u], acc.at[dst_sl.at[b, k]], ssem, add=True)
        return carry

    lax.fori_loop(0, NCHUNK // RING, step, 0)
    pltpu.make_async_copy(rows.at[(NCHUNK - 1) % RING],
                          acc.at[dst_sl.at[0, 0]], ssem).wait()
    plsc.subcore_barrier()
    pltpu.sync_copy(acc.at[pl.ds(s * RPT, RPT)], out_hbm.at[c, pl.ds(s * RPT, RPT)])


_ea_call = functools.partial(
    pl.kernel,
    mesh=plsc.VectorSubcoreMesh(core_axis_name="c", subcore_axis_name="s",
                                num_cores=NC, num_subcores=NS),
    out_type=jax.ShapeDtypeStruct((NC, NP, H), _f32),
    scratch_types=[
        pltpu.VMEM((2, SLOTC, CH), jnp.int32),
        pltpu.VMEM((RING, CH, H), _f32),
        pltpu.VMEM_SHARED((NP, H), _f32),
        pltpu.SemaphoreType.DMA,
        pltpu.SemaphoreType.DMA,
        pltpu.SemaphoreType.DMA,
        pltpu.SemaphoreType.DMA,
        pltpu.SemaphoreType.DMA,
        pltpu.SemaphoreType.DMA,
    ],
)(_ea_body)


def _wprep_kernel(wcat_ref, wpost_ref, o_ref):
    o_ref[0] = jnp.dot(wcat_ref[0], wpost_ref[0], preferred_element_type=_f32)


def _f0_kernel(x_ref, w_ref, b_ref, o_ref):
    o_ref[...] = jnp.maximum(
        jnp.dot(x_ref[...], w_ref[...], preferred_element_type=_f32) + b_ref[...], 0.0)


def _easum_kernel(ea_ref, o_ref):
    o_ref[...] = ea_ref[0, :, :2 * DE] + ea_ref[1, :, :2 * DE]


def _layer_kernel(sp_ref, f_ref, ea_ref, wa_ref, wb_ref, wp_ref, wc_ref, bp_ref, o_ref):
    sblk = sp_ref[0] + sp_ref[1]
    ea = ea_ref[...]
    deg = ea[:, DE:DE + 1]
    f = f_ref[...]
    o_ref[...] = (
        jnp.dot(sblk, wa_ref[...], preferred_element_type=_f32)
        + jnp.dot(deg * f, wb_ref[...], preferred_element_type=_f32)
        + jnp.dot(f, wp_ref[...], preferred_element_type=_f32)
        + jnp.dot(ea, wc_ref[...], preferred_element_type=_f32)
        + bp_ref[...])


def _readout_kernel(f_ref, w1a_ref, w1b_ref, w1c_ref, b1_ref, w2_ref, b2_ref, o_ref):
    f = f_ref[...]
    fs = jnp.sum(f, axis=0, keepdims=True)
    fm = jnp.max(f, axis=0, keepdims=True)
    h = jnp.maximum(
        jnp.dot(fs, w1a_ref[...], preferred_element_type=_f32)
        + jnp.dot(fs * (1.0 / N), w1b_ref[...], preferred_element_type=_f32)
        + jnp.dot(fm, w1c_ref[...], preferred_element_type=_f32)
        + b1_ref[...], 0.0)
    o_ref[...] = jnp.dot(h, w2_ref[...], preferred_element_type=_f32) + b2_ref[...]


_BN = 1000


def _tc_layer_call(sp, f, ea, wa, wb, wp, wc, bp):
    return pl.pallas_call(
        _layer_kernel,
        grid=(N // _BN,),
        in_specs=[
            pl.BlockSpec((NC, _BN, H), lambda i: (0, i, 0)),
            pl.BlockSpec((_BN, H), lambda i: (i, 0)),
            pl.BlockSpec((_BN, 2 * DE), lambda i: (i, 0)),
            pl.BlockSpec((H, H), lambda i: (0, 0)),
            pl.BlockSpec((H, H), lambda i: (0, 0)),
            pl.BlockSpec((H, H), lambda i: (0, 0)),
            pl.BlockSpec((2 * DE, H), lambda i: (0, 0)),
            pl.BlockSpec((1, H), lambda i: (0, 0)),
        ],
        out_specs=pl.BlockSpec((_BN, H), lambda i: (i, 0)),
        out_shape=jax.ShapeDtypeStruct((N, H), _f32),
    )(sp, f, ea, wa, wb, wp, wc, bp)


def kernel(x, edge_index, edge_attr, W_in, b_in, W_pre, b_pre, W_post, b_post,
           W_o1, b_o1, W_o2, b_o2):
    Wcat = jnp.concatenate(
        [W_pre, b_pre[:, None, :], jnp.zeros((L, 7, H), _f32)], axis=1)
    Weff = pl.pallas_call(
        _wprep_kernel,
        grid=(L,),
        in_specs=[pl.BlockSpec((1, 2 * H + DE + 8, H), lambda i: (i, 0, 0)),
                  pl.BlockSpec((1, H, H), lambda i: (i, 0, 0))],
        out_specs=pl.BlockSpec((1, 2 * H + DE + 8, H), lambda i: (i, 0, 0)),
        out_shape=jax.ShapeDtypeStruct((L, 2 * H + DE + 8, H), _f32),
    )(Wcat, W_post)
    Wa = Weff[:, :H]
    Wb = Weff[:, H:2 * H]
    Wc1 = jnp.concatenate(
        [Weff[:, 2 * H:2 * H + DE], Weff[:, 2 * H + DE:2 * H + DE + 1],
         jnp.zeros((L, DE - 1, H), _f32)], axis=1)

    ea1 = jnp.concatenate(
        [edge_attr, jnp.ones((E, 1), _f32), jnp.zeros((E, H - DE - 1), _f32)], axis=1)
    z128 = jnp.zeros((RPT, H), _f32)
    b_in2 = b_in[None, :]
    b_post2 = b_post[:, None, :]
    W1a, W1b, W1c = W_o1[:H], W_o1[H:2 * H], W_o1[2 * H:]
    b1 = b_o1[None, :]
    W2p = jnp.zeros((H, H), _f32).at[:, :T].set(W_o2)
    b2p = jnp.zeros((1, H), _f32).at[0, :T].set(b_o2)

    src_arr = edge_index[0].reshape(NW, NCHUNK, CH)
    dst_arr = edge_index[1].reshape(NW, NCHUNK, CH)
    ea3 = ea1.reshape(E // CH, CH, H)

    ea_part = _ea_call(ea3, dst_arr, z128)

    f = pl.pallas_call(
        _f0_kernel,
        grid=(N // _BN,),
        in_specs=[pl.BlockSpec((_BN, D), lambda i: (i, 0)),
                  pl.BlockSpec((D, H), lambda i: (0, 0)),
                  pl.BlockSpec((1, H), lambda i: (0, 0))],
        out_specs=pl.BlockSpec((_BN, H), lambda i: (i, 0)),
        out_shape=jax.ShapeDtypeStruct((N, H), _f32),
    )(x, W_in, b_in2)

    ea_s = pl.pallas_call(
        _easum_kernel,
        grid=(NP // 1024,),
        in_specs=[pl.BlockSpec((NC, 1024, H), lambda i: (0, i, 0))],
        out_specs=pl.BlockSpec((1024, 2 * DE), lambda i: (i, 0)),
        out_shape=jax.ShapeDtypeStruct((NP, 2 * DE), _f32),
    )(ea_part)

    for i in range(L):
        sp = _spmm_call(f, src_arr, dst_arr, z128)
        f = _tc_layer_call(sp, f, ea_s, Wa[i], Wb[i], W_post[i], Wc1[i],
                           b_post2[i])

    out = pl.pallas_call(
        _readout_kernel,
        in_specs=[pl.BlockSpec((N, H), lambda: (0, 0))] +
                 [pl.BlockSpec((H, H), lambda: (0, 0))] * 3 +
                 [pl.BlockSpec((1, H), lambda: (0, 0)),
                  pl.BlockSpec((H, H), lambda: (0, 0)),
                  pl.BlockSpec((1, H), lambda: (0, 0))],
        out_specs=pl.BlockSpec((1, H), lambda: (0, 0)),
        out_shape=jax.ShapeDtypeStruct((1, H), _f32),
    )(f, W1a, W1b, W1c, b1, W2p, b2p)
    return out[:, :T]


# speedup vs baseline: 1.1092x; 1.1092x over previous
import functools

import jax
import jax.numpy as jnp
from jax import lax
from jax.experimental import pallas as pl
from jax.experimental.pallas import tpu as pltpu
from jax.experimental.pallas import tpu_sc as plsc

N, E, D, DE, H, T, L = 10000, 320000, 128, 16, 128, 32, 5
NC, NS = 2, 16
NW = NC * NS
EPW = E // NW
NP = 10240
RPT = NP // NS

_f32 = jnp.float32


CH = 50
NCHUNK = EPW // CH
SLOTC = 8
NSLOT = NCHUNK // SLOTC
RING = 4
ECH = 40
ENCHUNK = EPW // ECH


def _spmm_body(f_hbm, src_hbm, dst_hbm, z_hbm, out_hbm, src_sl, dst_sl, rows,
               acc, isem, gsem0, gsem1, gsem2, gsem3, ssem):
    gsem = (gsem0, gsem1, gsem2, gsem3)
    c = lax.axis_index("c")
    s = lax.axis_index("s")
    wid = c * NS + s
    pltpu.sync_copy(src_hbm.at[wid, pl.ds(0, SLOTC)], src_sl.at[0])
    pltpu.sync_copy(dst_hbm.at[wid, pl.ds(0, SLOTC)], dst_sl.at[0])
    pltpu.sync_copy(z_hbm, acc.at[pl.ds(s * RPT, RPT)])
    plsc.subcore_barrier()

    for jj in range(RING - 1):
        pltpu.async_copy(f_hbm.at[src_sl.at[0, jj]], rows.at[jj], gsem[jj])

    def step(j0, carry):
        for u in range(RING):
            j = j0 * RING + u
            g = lax.div(j, SLOTC)
            k = lax.rem(j, SLOTC)
            b = lax.rem(g, 2)

            @pl.when(j >= 1)
            def _(u=u):
                pltpu.make_async_copy(rows.at[(u + RING - 1) % RING],
                                      acc.at[dst_sl.at[0, 0]], ssem).wait()

            @pl.when(jnp.logical_and(k == 0, j + SLOTC < NCHUNK))
            def _(g=g):
                bn = lax.rem(g + 1, 2)
                pltpu.async_copy(src_hbm.at[wid, pl.ds((g + 1) * SLOTC, SLOTC)],
                                 src_sl.at[bn], isem)
                pltpu.async_copy(dst_hbm.at[wid, pl.ds((g + 1) * SLOTC, SLOTC)],
                                 dst_sl.at[bn], isem)

            @pl.when(j + RING - 1 < NCHUNK)
            def _(j=j, u=u):
                j3 = j + RING - 1
                g3 = lax.div(j3, SLOTC)
                k3 = lax.rem(j3, SLOTC)
                b3 = lax.rem(g3, 2)

                @pl.when(k3 == 0)
                def _():
                    pltpu.make_async_copy(
                        src_hbm.at[wid, pl.ds(g3 * SLOTC, SLOTC)],
                        src_sl.at[b3], isem).wait()
                    pltpu.make_async_copy(
                        dst_hbm.at[wid, pl.ds(g3 * SLOTC, SLOTC)],
                        dst_sl.at[b3], isem).wait()

                pltpu.async_copy(f_hbm.at[src_sl.at[b3, k3]],
                                 rows.at[(u + RING - 1) % RING],
                                 gsem[(u + RING - 1) % RING])

            pltpu.make_async_copy(f_hbm.at[src_sl.at[b, k]],
                                  rows.at[u], gsem[u]).wait()
            pltpu.async_copy(rows.at[u], acc.at[dst_sl.at[b, k]], ssem, add=True)
        return carry

    lax.fori_loop(0, NCHUNK // RING, step, 0)
    pltpu.make_async_copy(rows.at[(NCHUNK - 1) % RING],
                          acc.at[dst_sl.at[0, 0]], ssem).wait()
    plsc.subcore_barrier()
    pltpu.sync_copy(acc.at[pl.ds(s * RPT, RPT)], out_hbm.at[c, pl.ds(s * RPT, RPT)])


_spmm_call = functools.partial(
    pl.kernel,
    mesh=plsc.VectorSubcoreMesh(core_axis_name="c", subcore_axis_name="s",
                                num_cores=NC, num_subcores=NS),
    out_type=jax.ShapeDtypeStruct((NC, NP, H), _f32),
    scratch_types=[
        pltpu.VMEM((2, SLOTC, CH), jnp.int32),
        pltpu.VMEM((2, SLOTC, CH), jnp.int32),
        pltpu.VMEM((RING, CH, H), _f32),
        pltpu.VMEM_SHARED((NP, H), _f32),
        pltpu.SemaphoreType.DMA,
        pltpu.SemaphoreType.DMA,
        pltpu.SemaphoreType.DMA,
        pltpu.SemaphoreType.DMA,
        pltpu.SemaphoreType.DMA,
        pltpu.SemaphoreType.DMA,
    ],
)(_spmm_body)


def _ea_body(ea_hbm, dst_hbm, z_hbm, out_hbm, dst2d, aug2, acc, gsem0, gsem1, ssem):
    gsem = (gsem0, gsem1)
    c = lax.axis_index("c")
    s = lax.axis_index("s")
    wid = c * NS + s
    base = wid * EPW
    pltpu.sync_copy(dst_hbm.at[wid], dst2d)
    pltpu.sync_copy(z_hbm, acc.at[pl.ds(s * RPT, RPT)])
    plsc.subcore_barrier()

    pltpu.async_copy(ea_hbm.at[pl.ds(base, ECH)], aug2.at[0], gsem[0])

    def chunkpair(j0, carry):
        for u in range(2):
            j = j0 * 2 + u

            @pl.when(j >= 1)
            def _(u=u):
                pltpu.make_async_copy(aug2.at[(u + 1) % 2],
                                      acc.at[dst2d.at[0]], ssem).wait()

            @pl.when(j < ENCHUNK - 1)
            def _(j=j, u=u):
                pltpu.async_copy(ea_hbm.at[pl.ds(base + (j + 1) * ECH, ECH)],
                                 aug2.at[(u + 1) % 2], gsem[(u + 1) % 2])

            pltpu.make_async_copy(ea_hbm.at[pl.ds(base + j * ECH, ECH)],
                                  aug2.at[u], gsem[u]).wait()
            pltpu.async_copy(aug2.at[u], acc.at[dst2d.at[j]], ssem, add=True)
        return carry
    lax.fori_loop(0, ENCHUNK // 2, chunkpair, 0)
    pltpu.make_async_copy(aug2.at[(ENCHUNK - 1) % 2],
                          acc.at[dst2d.at[0]], ssem).wait()

    plsc.subcore_barrier()
    pltpu.sync_copy(acc.at[pl.ds(s * RPT, RPT)], out_hbm.at[c, pl.ds(s * RPT, RPT)])


_ea_call = functools.partial(
    pl.kernel,
    mesh=plsc.VectorSubcoreMesh(core_axis_name="c", subcore_axis_name="s",
                                num_cores=NC, num_subcores=NS),
    out_type=jax.ShapeDtypeStruct((NC, NP, H), _f32),
    scratch_types=[
        pltpu.VMEM((ENCHUNK, ECH), jnp.int32),
        pltpu.VMEM((2, ECH, H), _f32),
        pltpu.VMEM_SHARED((NP, H), _f32),
        pltpu.SemaphoreType.DMA,
        pltpu.SemaphoreType.DMA,
        pltpu.SemaphoreType.DMA,
    ],
)(_ea_body)


def _wprep_kernel(wcat_ref, wpost_ref, o_ref):
    o_ref[0] = jnp.dot(wcat_ref[0], wpost_ref[0], preferred_element_type=_f32)


def _f0_kernel(x_ref, w_ref, b_ref, o_ref):
    o_ref[...] = jnp.maximum(
        jnp.dot(x_ref[...], w_ref[...], preferred_element_type=_f32) + b_ref[...], 0.0)


def _layer_kernel(sp_ref, f_ref, ea_ref, wa_ref, wb_ref, wp_ref, wc_ref, bp_ref, o_ref):
    sblk = sp_ref[0] + sp_ref[1]
    ea = ea_ref[0] + ea_ref[1]
    deg = ea[:, DE:DE + 1]
    f = f_ref[...]
    o_ref[...] = (
        jnp.dot(sblk, wa_ref[...], preferred_element_type=_f32)
        + jnp.dot(deg * f, wb_ref[...], preferred_element_type=_f32)
        + jnp.dot(f, wp_ref[...], preferred_element_type=_f32)
        + jnp.dot(ea, wc_ref[...], preferred_element_type=_f32)
        + bp_ref[...])


def _readout_kernel(f_ref, w1a_ref, w1b_ref, w1c_ref, b1_ref, w2_ref, b2_ref, o_ref):
    f = f_ref[...]
    fs = jnp.sum(f, axis=0, keepdims=True)
    fm = jnp.max(f, axis=0, keepdims=True)
    h = jnp.maximum(
        jnp.dot(fs, w1a_ref[...], preferred_element_type=_f32)
        + jnp.dot(fs * (1.0 / N), w1b_ref[...], preferred_element_type=_f32)
        + jnp.dot(fm, w1c_ref[...], preferred_element_type=_f32)
        + b1_ref[...], 0.0)
    o_ref[...] = jnp.dot(h, w2_ref[...], preferred_element_type=_f32) + b2_ref[...]


_BN = 1000


def _tc_layer_call(sp, f, ea, wa, wb, wp, wc, bp):
    return pl.pallas_call(
        _layer_kernel,
        grid=(N // _BN,),
        in_specs=[
            pl.BlockSpec((NC, _BN, H), lambda i: (0, i, 0)),
            pl.BlockSpec((_BN, H), lambda i: (i, 0)),
            pl.BlockSpec((NC, _BN, H), lambda i: (0, i, 0)),
            pl.BlockSpec((H, H), lambda i: (0, 0)),
            pl.BlockSpec((H, H), lambda i: (0, 0)),
            pl.BlockSpec((H, H), lambda i: (0, 0)),
            pl.BlockSpec((H, H), lambda i: (0, 0)),
            pl.BlockSpec((1, H), lambda i: (0, 0)),
        ],
        out_specs=pl.BlockSpec((_BN, H), lambda i: (i, 0)),
        out_shape=jax.ShapeDtypeStruct((N, H), _f32),
    )(sp, f, ea, wa, wb, wp, wc, bp)


def kernel(x, edge_index, edge_attr, W_in, b_in, W_pre, b_pre, W_post, b_post,
           W_o1, b_o1, W_o2, b_o2):
    Wcat = jnp.concatenate(
        [W_pre, b_pre[:, None, :], jnp.zeros((L, 7, H), _f32)], axis=1)
    Weff = pl.pallas_call(
        _wprep_kernel,
        grid=(L,),
        in_specs=[pl.BlockSpec((1, 2 * H + DE + 8, H), lambda i: (i, 0, 0)),
                  pl.BlockSpec((1, H, H), lambda i: (i, 0, 0))],
        out_specs=pl.BlockSpec((1, 2 * H + DE + 8, H), lambda i: (i, 0, 0)),
        out_shape=jax.ShapeDtypeStruct((L, 2 * H + DE + 8, H), _f32),
    )(Wcat, W_post)
    Wa = Weff[:, :H]
    Wb = Weff[:, H:2 * H]
    Wc1 = jnp.concatenate(
        [Weff[:, 2 * H:2 * H + DE], Weff[:, 2 * H + DE:2 * H + DE + 1],
         jnp.zeros((L, H - DE - 1, H), _f32)], axis=1)

    ea1 = jnp.concatenate(
        [edge_attr, jnp.ones((E, 1), _f32), jnp.zeros((E, H - DE - 1), _f32)], axis=1)
    z128 = jnp.zeros((RPT, H), _f32)
    b_in2 = b_in[None, :]
    b_post2 = b_post[:, None, :]
    W1a, W1b, W1c = W_o1[:H], W_o1[H:2 * H], W_o1[2 * H:]
    b1 = b_o1[None, :]
    W2p = jnp.zeros((H, H), _f32).at[:, :T].set(W_o2)
    b2p = jnp.zeros((1, H), _f32).at[0, :T].set(b_o2)

    src_arr = edge_index[0].reshape(NW, NCHUNK, CH)
    dst_arr = edge_index[1].reshape(NW, NCHUNK, CH)
    dst_ea = edge_index[1].reshape(NW, ENCHUNK, ECH)

    ea_part = _ea_call(ea1, dst_ea, z128)

    f = pl.pallas_call(
        _f0_kernel,
        grid=(N // _BN,),
        in_specs=[pl.BlockSpec((_BN, D), lambda i: (i, 0)),
                  pl.BlockSpec((D, H), lambda i: (0, 0)),
                  pl.BlockSpec((1, H), lambda i: (0, 0))],
        out_specs=pl.BlockSpec((_BN, H), lambda i: (i, 0)),
        out_shape=jax.ShapeDtypeStruct((N, H), _f32),
    )(x, W_in, b_in2)

    for i in range(L):
        sp = _spmm_call(f, src_arr, dst_arr, z128)
        f = _tc_layer_call(sp, f, ea_part, Wa[i], Wb[i], W_post[i], Wc1[i],
                           b_post2[i])

    out = pl.pallas_call(
        _readout_kernel,
        in_specs=[pl.BlockSpec((N, H), lambda: (0, 0))] +
                 [pl.BlockSpec((H, H), lambda: (0, 0))] * 3 +
                 [pl.BlockSpec((1, H), lambda: (0, 0)),
                  pl.BlockSpec((H, H), lambda: (0, 0)),
                  pl.BlockSpec((1, H), lambda: (0, 0))],
        out_specs=pl.BlockSpec((1, H), lambda: (0, 0)),
        out_shape=jax.ShapeDtypeStruct((1, H), _f32),
    )(f, W1a, W1b, W1c, b1, W2p, b2p)
    return out[:, :T]
